# Initial kernel scaffold; baseline (speedup 1.0000x reference)
import jax
import jax.numpy as jnp
from jax.experimental import pallas as pl


def kernel(h, norm, prev_h, emb_rel, edge_index, edge_type, weight_neighbor, loop_weight, evolve_loop_weight, skip_connect_weight, skip_connect_bias):
    raise NotImplementedError("write your pallas kernel here")


# speedup vs baseline: 4.0420x; 4.0420x over previous
import functools

import jax
import jax.numpy as jnp
from jax import lax
from jax.experimental import pallas as pl
from jax.experimental.pallas import tpu as pltpu
from jax.experimental.pallas import tpu_sc as plsc

NC = 2
NS = 16
NW = NC * NS
K = 64
C = 16


def _sc_edge_kernel(n_pad, nchunk, h, emb_rel, src_r, dst_r, typ_r):
    D = h.shape[1]
    rows_per_tile = n_pad // NS
    assert rows_per_tile % K == 0
    nw_rows = rows_per_tile // K
    mesh = plsc.VectorSubcoreMesh(core_axis_name="c", subcore_axis_name="s")

    @functools.partial(
        pl.kernel,
        out_type=[
            jax.ShapeDtypeStruct((NC, n_pad, D), jnp.float32),
            jax.ShapeDtypeStruct((NC, n_pad, 16), jnp.float32),
        ],
        mesh=mesh,
        scratch_types=[
            pltpu.VMEM((C, K), jnp.int32),
            pltpu.VMEM((C, K), jnp.int32),
            pltpu.VMEM((C, K), jnp.int32),
            pltpu.VMEM((K, D), jnp.float32),
            pltpu.VMEM((K, D), jnp.float32),
            pltpu.VMEM((K, 16), jnp.float32),
            pltpu.VMEM((nw_rows, K), jnp.int32),
            pltpu.VMEM_SHARED((n_pad, D), jnp.float32),
            pltpu.VMEM_SHARED((n_pad, 16), jnp.float32),
            pltpu.SemaphoreType.DMA,
            pltpu.SemaphoreType.DMA,
        ],
    )
    def k(h_hbm, rel_hbm, src_hbm, dst_hbm, typ_hbm,
          acc_out, deg_out, src_v, dst_v, typ_v, hbuf, rbuf,
          ones_v, rows_idx, acc_sh, deg_sh, sem0, sem1):
        c = lax.axis_index("c")
        s = lax.axis_index("s")
        wid = s * NC + c
        r0 = s * rows_per_tile

        z16 = jnp.zeros((16,), dtype=jnp.float32)
        iota16 = lax.iota(jnp.int32, 16)

        def zrow(i, _):
            for t in range(D // 16):
                hbuf[i, pl.ds(t * 16, 16)] = z16
            ones_v[i] = z16
            return _

        lax.fori_loop(0, K, zrow, None)

        def idxrow(t, _):
            base = r0 + t * K
            for u in range(K // 16):
                rows_idx[t, pl.ds(u * 16, 16)] = iota16 + (base + u * 16)
            return _

        lax.fori_loop(0, nw_rows, idxrow, None)

        def zinit(t, _):
            pltpu.sync_copy(hbuf, acc_sh.at[rows_idx.at[t]])
            pltpu.sync_copy(ones_v, deg_sh.at[rows_idx.at[t]])
            return _

        lax.fori_loop(0, nw_rows, zinit, None)

        ones_row = jnp.full((16,), 1.0, dtype=jnp.float32)
        for i in range(K):
            ones_v[i] = ones_row

        plsc.subcore_barrier()

        def chunk_body(q, _):
            pltpu.sync_copy(src_hbm.at[wid, pl.ds(q * C, C)], src_v)
            pltpu.sync_copy(dst_hbm.at[wid, pl.ds(q * C, C)], dst_v)
            pltpu.sync_copy(typ_hbm.at[wid, pl.ds(q * C, C)], typ_v)

            def body(j, _):
                sidx = src_v.at[j]
                didx = dst_v.at[j]
                tidx = typ_v.at[j]
                cp0 = pltpu.async_copy(h_hbm.at[sidx], hbuf, sem0)
                cp1 = pltpu.async_copy(rel_hbm.at[tidx], rbuf, sem1)
                cp0.wait()
                cp1.wait()
                pltpu.sync_copy(hbuf, acc_sh.at[didx], add=True)
                pltpu.sync_copy(rbuf, acc_sh.at[didx], add=True)
                pltpu.sync_copy(ones_v, deg_sh.at[didx], add=True)
                return _

            lax.fori_loop(0, C, body, None)
            return _

        lax.fori_loop(0, nchunk, chunk_body, None)

        plsc.subcore_barrier()

        def wout(t, _):
            pltpu.sync_copy(acc_sh.at[rows_idx.at[t]], hbuf)
            pltpu.sync_copy(hbuf, acc_out.at[c, pl.ds(r0 + t * K, K)])
            pltpu.sync_copy(deg_sh.at[rows_idx.at[t]], ones_v)
            pltpu.sync_copy(ones_v, deg_out.at[c, pl.ds(r0 + t * K, K)])
            return _

        lax.fori_loop(0, nw_rows, wout, None)

    return k(h, emb_rel, src_r, dst_r, typ_r)


def _tc_dense_body(h_ref, norm_ref, prev_ref, acc0_ref, acc1_ref,
                   deg0_ref, deg1_ref, wn_ref, wl_ref, we_ref, ws_ref,
                   b_ref, out_ref):
    f32 = jnp.float32
    p = acc0_ref[...] + acc1_ref[...]
    agg = jnp.dot(p, wn_ref[...], preferred_element_type=f32)
    deg = jnp.sum(deg0_ref[...] + deg1_ref[...], axis=1, keepdims=True)
    h = h_ref[...]
    hl = jnp.dot(h, wl_ref[...], preferred_element_type=f32)
    he = jnp.dot(h, we_ref[...], preferred_element_type=f32)
    loop_msg = jnp.where(deg > 0.0, hl, he)
    prev = prev_ref[...]
    gate = jnp.dot(prev, ws_ref[...], preferred_element_type=f32) + b_ref[...]
    skip = 1.0 / (1.0 + jnp.exp(-gate))
    node = agg * norm_ref[...] + loop_msg
    out_ref[...] = skip * node + (1.0 - skip) * prev


def kernel(h, norm, prev_h, emb_rel, edge_index, edge_type,
           weight_neighbor, loop_weight, evolve_loop_weight,
           skip_connect_weight, skip_connect_bias):
    n, d = h.shape
    e = edge_index.shape[1]
    assert d == 128

    epw_raw = -(-e // NW)
    epw = -(-epw_raw // (C * K)) * (C * K)
    nb = epw // K
    nchunk = nb // C
    e_pad = epw * NW
    n_pad = -(-(n + 1) // (NS * K)) * (NS * K)

    src = edge_index[0]
    dst = edge_index[1]
    pad = e_pad - e
    src_r = jnp.pad(src, (0, pad)).reshape(NW, nb, K)
    typ_r = jnp.pad(edge_type, (0, pad)).reshape(NW, nb, K)
    dst_r = jnp.pad(dst, (0, pad), constant_values=n).reshape(NW, nb, K)
    del nb

    acc, deg = _sc_edge_kernel(n_pad, nchunk, h, emb_rel, src_r,
                               dst_r, typ_r)

    bn = 2000
    grid = (n // bn,)
    row_block = lambda w: pl.BlockSpec((bn, w), lambda i: (i, 0))
    full = lambda a, b: pl.BlockSpec((a, b), lambda i: (0, 0))
    out = pl.pallas_call(
        _tc_dense_body,
        grid=grid,
        in_specs=[
            row_block(d),
            row_block(1),
            row_block(d),
            row_block(d),
            row_block(d),
            row_block(16),
            row_block(16),
            full(d, d), full(d, d), full(d, d), full(d, d),
            full(1, d),
        ],
        out_specs=row_block(d),
        out_shape=jax.ShapeDtypeStruct((n, d), jnp.float32),
    )(h, norm, prev_h, acc[0, :n], acc[1, :n], deg[0, :n], deg[1, :n],
      weight_neighbor, loop_weight, evolve_loop_weight, skip_connect_weight,
      skip_connect_bias.reshape(1, d))
    return out
